# baseline (speedup 1.0000x reference)
import functools

import jax
import jax.numpy as jnp
from jax import lax
from jax.experimental import pallas as pl
from jax.experimental.pallas import tpu as pltpu
from jax.experimental.pallas import tpu_sc as plsc

_BATCH = 16384
_EMB = 64

_info = plsc.get_sparse_core_info()
_NC = _info.num_cores
_NS = _info.num_subcores
_NW = _NC * _NS
_BPW = _BATCH // _NW
_CH = 128
_NCH = _BPW // _CH

_sc_mesh = plsc.VectorSubcoreMesh(core_axis_name="c", subcore_axis_name="s")


@functools.partial(
    pl.kernel,
    out_type=(
        jax.ShapeDtypeStruct((_BATCH, _EMB), jnp.float32),
        jax.ShapeDtypeStruct((_BATCH, _EMB), jnp.float32),
    ),
    mesh=_sc_mesh,
    scratch_types=[
        pltpu.VMEM((_BPW + 16,), jnp.int32),
        pltpu.VMEM((_BPW + 16,), jnp.int32),
        pltpu.SemaphoreType.DMA,
    ],
)
def _sc_gather(uid_hbm, iid_hbm, utab_hbm, itab_hbm, u_out, i_out,
               uidx, iidx, sem):
    wid = lax.axis_index("s") * _NC + lax.axis_index("c")
    base = wid * _BPW
    pltpu.sync_copy(uid_hbm.at[pl.ds(base, _BPW)], uidx.at[pl.ds(0, _BPW)])
    pltpu.sync_copy(iid_hbm.at[pl.ds(base, _BPW)], iidx.at[pl.ds(0, _BPW)])

    def row_copies(c, tab, idx_s, out):
        def body(j, _):
            r = idx_s[pl.ds(c * _CH + j, 16)][0]
            pltpu.async_copy(tab.at[pl.ds(r, 1)],
                             out.at[pl.ds(base + c * _CH + j, 1)], sem)
            return _
        lax.fori_loop(0, _CH, body, 0)

    for c in range(_NCH):
        row_copies(c, utab_hbm, uidx, u_out)
        row_copies(c, itab_hbm, iidx, i_out)
    for c in range(_NCH):
        pltpu.make_async_copy(
            utab_hbm.at[pl.ds(0, _CH)],
            u_out.at[pl.ds(base + c * _CH, _CH)], sem).wait()
        pltpu.make_async_copy(
            itab_hbm.at[pl.ds(0, _CH)],
            i_out.at[pl.ds(base + c * _CH, _CH)], sem).wait()


_ROWS = 2048


def _mlp_body(u_ref, i_ref, wlin_ref, w1_ref, w2_ref, w3_ref,
              b1_ref, b2_ref, scal_ref, o_ref):
    u = u_ref[...]
    it = i_ref[...]
    w1 = w1_ref[...]
    h1 = jnp.dot(u, w1[:_EMB], preferred_element_type=jnp.float32)
    h1 = h1 + jnp.dot(it, w1[_EMB:], preferred_element_type=jnp.float32)
    h1 = jnp.maximum(h1 + b1_ref[...], 0.0)
    h2 = jnp.dot(h1, w2_ref[...], preferred_element_type=jnp.float32)
    h2 = jnp.maximum(h2 + b2_ref[...], 0.0)
    neural = jnp.dot(h2, w3_ref[...], preferred_element_type=jnp.float32)
    wlin = wlin_ref[...]
    linear = jnp.dot(u, wlin[:_EMB], preferred_element_type=jnp.float32)
    linear = linear + jnp.dot(it, wlin[_EMB:], preferred_element_type=jnp.float32)
    b_lin = scal_ref[0, 0]
    b3 = scal_ref[0, 1]
    g = jax.nn.sigmoid(scal_ref[0, 2])
    o_ref[...] = g * (linear + b_lin) + (1.0 - g) * (neural + b3)


def _mlp_call(u_emb, i_emb, W_lin, W1, W2, W3, b1, b2, scalars):
    n_blocks = _BATCH // _ROWS
    full = lambda shape: pl.BlockSpec(shape, lambda i: (0,) * len(shape))
    return pl.pallas_call(
        _mlp_body,
        grid=(n_blocks,),
        in_specs=[
            pl.BlockSpec((_ROWS, _EMB), lambda i: (i, 0)),
            pl.BlockSpec((_ROWS, _EMB), lambda i: (i, 0)),
            full((2 * _EMB, 1)),
            full((2 * _EMB, _EMB)),
            full((_EMB, 32)),
            full((32, 1)),
            full((1, _EMB)),
            full((1, 32)),
            full((1, 3)),
        ],
        out_specs=pl.BlockSpec((_ROWS, 1), lambda i: (i, 0)),
        out_shape=jax.ShapeDtypeStruct((_BATCH, 1), jnp.float32),
        compiler_params=pltpu.CompilerParams(
            dimension_semantics=("arbitrary",),
        ),
    )(u_emb, i_emb, W_lin, W1, W2, W3, b1, b2, scalars)


def kernel(user_ids, item_ids, user_table, item_table,
           W_lin, b_lin, W1, b1, W2, b2, W3, b3, alpha):
    u_emb, i_emb = _sc_gather(user_ids, item_ids, user_table, item_table)
    scalars = jnp.stack([b_lin[0], b3[0], alpha[0]]).reshape(1, 3)
    return _mlp_call(u_emb, i_emb, W_lin, W1, W2, W3,
                     b1.reshape(1, _EMB), b2.reshape(1, 32), scalars)


# speedup vs baseline: 1.6768x; 1.6768x over previous
import functools

import jax
import jax.numpy as jnp
from jax import lax
from jax.experimental import pallas as pl
from jax.experimental.pallas import tpu as pltpu
from jax.experimental.pallas import tpu_sc as plsc

_BATCH = 16384
_EMB = 64

_info = plsc.get_sparse_core_info()
_NC = _info.num_cores
_NS = _info.num_subcores
_NW = _NC * _NS
_BPW = _BATCH // _NW
_CH = 256
_NCH = _BPW // _CH

_sc_mesh = plsc.VectorSubcoreMesh(core_axis_name="c", subcore_axis_name="s")


@functools.partial(
    pl.kernel,
    out_type=(
        jax.ShapeDtypeStruct((_BATCH, _EMB), jnp.float32),
        jax.ShapeDtypeStruct((_BATCH, _EMB), jnp.float32),
    ),
    mesh=_sc_mesh,
    scratch_types=[
        pltpu.VMEM((_BPW,), jnp.int32),
        pltpu.VMEM((_BPW,), jnp.int32),
        pltpu.VMEM((_CH, _EMB), jnp.float32),
        pltpu.VMEM((_CH, _EMB), jnp.float32),
        pltpu.SemaphoreType.DMA,
        pltpu.SemaphoreType.DMA,
    ],
)
def _sc_gather(uid_hbm, iid_hbm, utab_hbm, itab_hbm, u_out, i_out,
               uidx, iidx, rbuf0, rbuf1, gsem, osem):
    wid = lax.axis_index("s") * _NC + lax.axis_index("c")
    base = wid * _BPW
    pltpu.sync_copy(uid_hbm.at[pl.ds(base, _BPW)], uidx)
    pltpu.sync_copy(iid_hbm.at[pl.ds(base, _BPW)], iidx)

    rbufs = (rbuf0, rbuf1)
    work = []
    for t, (tab, idx, out) in enumerate(
        ((utab_hbm, uidx, u_out), (itab_hbm, iidx, i_out))):
        for c in range(_NCH):
            work.append((tab, idx, out, c, rbufs[(t * _NCH + c) % 2]))

    def fire(w):
        tab, idx, out, c, buf = w

        def body(g, _):
            v = idx[pl.ds(c * _CH + g * 16, 16)]
            for k in range(16):
                pltpu.async_copy(tab.at[pl.ds(v[k], 1)],
                                 buf.at[pl.ds(g * 16 + k, 1)], gsem)
            return _
        lax.fori_loop(0, _CH // 16, body, 0)

    def drain(w):
        tab, idx, out, c, buf = w
        pltpu.make_async_copy(tab.at[pl.ds(0, _CH)], buf, gsem).wait()

    def flush(w):
        tab, idx, out, c, buf = w
        pltpu.async_copy(buf, out.at[pl.ds(base + c * _CH, _CH)], osem)

    def drain_flush(w):
        tab, idx, out, c, buf = w
        pltpu.make_async_copy(buf, out.at[pl.ds(base + c * _CH, _CH)],
                              osem).wait()

    n = len(work)
    fire(work[0])
    for i in range(n):
        if i + 1 < n:
            fire(work[i + 1])
        drain(work[i])
        if i >= 1:
            drain_flush(work[i - 1])
        flush(work[i])
    drain_flush(work[n - 1])


_ROWS = 2048


def _mlp_body(u_ref, i_ref, wlin_ref, w1_ref, w2_ref, w3_ref,
              b1_ref, b2_ref, scal_ref, o_ref):
    u = u_ref[...]
    it = i_ref[...]
    w1 = w1_ref[...]
    h1 = jnp.dot(u, w1[:_EMB], preferred_element_type=jnp.float32)
    h1 = h1 + jnp.dot(it, w1[_EMB:], preferred_element_type=jnp.float32)
    h1 = jnp.maximum(h1 + b1_ref[...], 0.0)
    h2 = jnp.dot(h1, w2_ref[...], preferred_element_type=jnp.float32)
    h2 = jnp.maximum(h2 + b2_ref[...], 0.0)
    neural = jnp.dot(h2, w3_ref[...], preferred_element_type=jnp.float32)
    wlin = wlin_ref[...]
    linear = jnp.dot(u, wlin[:_EMB], preferred_element_type=jnp.float32)
    linear = linear + jnp.dot(it, wlin[_EMB:], preferred_element_type=jnp.float32)
    b_lin = scal_ref[0, 0]
    b3 = scal_ref[0, 1]
    g = jax.nn.sigmoid(scal_ref[0, 2])
    o_ref[...] = g * (linear + b_lin) + (1.0 - g) * (neural + b3)


def _mlp_call(u_emb, i_emb, W_lin, W1, W2, W3, b1, b2, scalars):
    n_blocks = _BATCH // _ROWS
    full = lambda shape: pl.BlockSpec(shape, lambda i: (0,) * len(shape))
    return pl.pallas_call(
        _mlp_body,
        grid=(n_blocks,),
        in_specs=[
            pl.BlockSpec((_ROWS, _EMB), lambda i: (i, 0)),
            pl.BlockSpec((_ROWS, _EMB), lambda i: (i, 0)),
            full((2 * _EMB, 1)),
            full((2 * _EMB, _EMB)),
            full((_EMB, 32)),
            full((32, 1)),
            full((1, _EMB)),
            full((1, 32)),
            full((1, 3)),
        ],
        out_specs=pl.BlockSpec((_ROWS, 1), lambda i: (i, 0)),
        out_shape=jax.ShapeDtypeStruct((_BATCH, 1), jnp.float32),
        compiler_params=pltpu.CompilerParams(
            dimension_semantics=("arbitrary",),
        ),
    )(u_emb, i_emb, W_lin, W1, W2, W3, b1, b2, scalars)


def kernel(user_ids, item_ids, user_table, item_table,
           W_lin, b_lin, W1, b1, W2, b2, W3, b3, alpha):
    u_emb, i_emb = _sc_gather(user_ids, item_ids, user_table, item_table)
    scalars = jnp.stack([b_lin[0], b3[0], alpha[0]]).reshape(1, 3)
    return _mlp_call(u_emb, i_emb, W_lin, W1, W2, W3,
                     b1.reshape(1, _EMB), b2.reshape(1, 32), scalars)


# speedup vs baseline: 2.0244x; 1.2073x over previous
import functools

import jax
import jax.numpy as jnp
from jax import lax
from jax.experimental import pallas as pl
from jax.experimental.pallas import tpu as pltpu
from jax.experimental.pallas import tpu_sc as plsc

_BATCH = 16384
_EMB = 64
_NROWS = 1000000
_NPANELS = _NROWS // 128
_ALIGNED = _NPANELS * 128
_NTAIL = _NROWS - _ALIGNED

_info = plsc.get_sparse_core_info()
_NC = _info.num_cores
_NS = _info.num_subcores
_NW = _NC * _NS
_CAP = 1024
_RING = 32

_sc_mesh = plsc.VectorSubcoreMesh(core_axis_name="c", subcore_axis_name="s")


def _splat(x, n=16):
    return jnp.full((n,), x, jnp.int32)


@functools.partial(
    pl.kernel,
    out_type=(
        jax.ShapeDtypeStruct((_BATCH, _EMB), jnp.float32),
        jax.ShapeDtypeStruct((_BATCH, _EMB), jnp.float32),
    ),
    mesh=_sc_mesh,
    scratch_types=[
        pltpu.VMEM((_BATCH,), jnp.int32),
        pltpu.VMEM((_CAP + 16,), jnp.int32),
        pltpu.VMEM((_CAP + 16,), jnp.int32),
        pltpu.VMEM((2, _EMB, 128), jnp.float32),
        pltpu.VMEM((32,), jnp.int32),
        pltpu.VMEM((32,), jnp.int32),
        pltpu.VMEM((_RING, _EMB), jnp.float32),
        pltpu.SemaphoreType.DMA,
        pltpu.SemaphoreType.DMA,
    ],
    compiler_params=pltpu.CompilerParams(needs_layout_passes=False),
)
def _sc_gather(uid_hbm, iid_hbm, utab_hbm, itab_hbm, u_out, i_out,
               idxv, hj, hr, pbuf, tj, tr, ring, gsem, osem):
    wid = lax.axis_index("s") * _NC + lax.axis_index("c")
    iota = lax.iota(jnp.int32, 16)

    def one_table(tab, ids_hbm, out, total_in):
        pltpu.sync_copy(ids_hbm, idxv)

        def scan_body(g, off):
            rv = idxv[pl.ds(g * 16, 16)]
            rv = jnp.where(rv >= _ALIGNED, 0, rv)
            mask = ((rv >> 7) & 31) == wid
            jv = iota + g * 16
            csum = plsc.cumsum(mask.astype(jnp.int32))
            pos = off + csum - 1
            plsc.store_scatter(hj, [pos], jv, mask=mask)
            plsc.store_scatter(hr, [pos], rv, mask=mask)
            return off + csum[15]

        nhits = lax.fori_loop(0, _BATCH // 16, scan_body, 0)
        ngroups = (nhits + 15) // 16
        npan = (_NPANELS - wid + _NW - 1) // _NW

        def panel_start(k):
            return pl.multiple_of((wid + k * _NW) * 128, 128)

        def fire_panel(k, par):
            pltpu.async_copy(tab.at[:, pl.ds(panel_start(k), 128)],
                             pbuf.at[par], gsem)

        fire_panel(0, 0)

        def panel_body(k, total):
            par = lax.rem(k, 2)

            @pl.when(k + 1 < npan)
            def _():
                fire_panel(k + 1, 1 - par)

            pltpu.make_async_copy(tab.at[:, pl.ds(0, 128)],
                                  pbuf.at[0], gsem).wait()
            p = wid + k * _NW

            def group_body(m, total):
                rv = hr[pl.ds(m * 16, 16)]
                jv = hj[pl.ds(m * 16, 16)]
                lane = iota + m * 16
                msk = jnp.logical_and((rv >> 7) == p, lane < nhits)
                csum = plsc.cumsum(msk.astype(jnp.int32))
                cnt = csum[15]

                def extract(q, total):
                    j = tj[pl.ds(q, 16)][0]
                    rl = tr[pl.ds(q, 16)][0] & 127
                    slot = lax.rem(total, _RING)

                    @pl.when(total >= _RING)
                    def _():
                        pltpu.make_async_copy(
                            out.at[0], ring.at[0], osem).wait()
                    for c in range(_EMB // 16):
                        vals = plsc.load_gather(
                            pbuf, [_splat(par), iota + c * 16, _splat(rl)])
                        ring[slot, pl.ds(c * 16, 16)] = vals
                    pltpu.async_copy(ring.at[slot], out.at[j], osem)
                    return total + 1

                @pl.when(cnt > 0)
                def _():
                    pos = csum - 1
                    plsc.store_scatter(tj, [pos], jv, mask=msk)
                    plsc.store_scatter(tr, [pos], rv, mask=msk)

                return lax.cond(cnt > 0,
                                lambda t: lax.fori_loop(0, cnt, extract, t),
                                lambda t: t, total)

            return lax.fori_loop(0, ngroups, group_body, total)

        return lax.fori_loop(0, npan, panel_body, total_in)

    total = one_table(utab_hbm, uid_hbm, u_out, 0)
    total = one_table(itab_hbm, iid_hbm, i_out, total)

    def drain_body(q, _):
        pltpu.make_async_copy(u_out.at[0], ring.at[0], osem).wait()
        return _
    lax.fori_loop(0, jnp.minimum(total, _RING), drain_body, 0)


_ROWS = 2048


def _mlp_body(uid_ref, iid_ref, u_ref, i_ref, utail_ref, itail_ref,
              wlin_ref, w1_ref, w2_ref, w3_ref, b1_ref, b2_ref, scal_ref,
              o_ref):
    uid = uid_ref[...].reshape(_ROWS, 1)
    iid = iid_ref[...].reshape(_ROWS, 1)
    tail_iota = lax.broadcasted_iota(jnp.int32, (_ROWS, _NTAIL), 1)
    u_oh = (uid - _ALIGNED == tail_iota).astype(jnp.float32)
    i_oh = (iid - _ALIGNED == tail_iota).astype(jnp.float32)
    u_m = (uid >= _ALIGNED).astype(jnp.float32)
    i_m = (iid >= _ALIGNED).astype(jnp.float32)
    u = u_ref[...] * (1.0 - u_m) + jnp.dot(
        u_oh, utail_ref[...], preferred_element_type=jnp.float32)
    it = i_ref[...] * (1.0 - i_m) + jnp.dot(
        i_oh, itail_ref[...], preferred_element_type=jnp.float32)
    w1 = w1_ref[...]
    h1 = jnp.dot(u, w1[:_EMB], preferred_element_type=jnp.float32)
    h1 = h1 + jnp.dot(it, w1[_EMB:], preferred_element_type=jnp.float32)
    h1 = jnp.maximum(h1 + b1_ref[...], 0.0)
    h2 = jnp.dot(h1, w2_ref[...], preferred_element_type=jnp.float32)
    h2 = jnp.maximum(h2 + b2_ref[...], 0.0)
    neural = jnp.dot(h2, w3_ref[...], preferred_element_type=jnp.float32)
    wlin = wlin_ref[...]
    linear = jnp.dot(u, wlin[:_EMB], preferred_element_type=jnp.float32)
    linear = linear + jnp.dot(it, wlin[_EMB:], preferred_element_type=jnp.float32)
    b_lin = scal_ref[0, 0]
    b3 = scal_ref[0, 1]
    g = jax.nn.sigmoid(scal_ref[0, 2])
    o_ref[...] = g * (linear + b_lin) + (1.0 - g) * (neural + b3)


def _mlp_call(uid, iid, u_emb, i_emb, utail, itail,
              W_lin, W1, W2, W3, b1, b2, scalars):
    n_blocks = _BATCH // _ROWS
    full = lambda shape: pl.BlockSpec(shape, lambda i: (0,) * len(shape))
    return pl.pallas_call(
        _mlp_body,
        grid=(n_blocks,),
        in_specs=[
            pl.BlockSpec((_ROWS,), lambda i: (i,)),
            pl.BlockSpec((_ROWS,), lambda i: (i,)),
            pl.BlockSpec((_ROWS, _EMB), lambda i: (i, 0)),
            pl.BlockSpec((_ROWS, _EMB), lambda i: (i, 0)),
            full((_NTAIL, _EMB)),
            full((_NTAIL, _EMB)),
            full((2 * _EMB, 1)),
            full((2 * _EMB, _EMB)),
            full((_EMB, 32)),
            full((32, 1)),
            full((1, _EMB)),
            full((1, 32)),
            full((1, 3)),
        ],
        out_specs=pl.BlockSpec((_ROWS, 1), lambda i: (i, 0)),
        out_shape=jax.ShapeDtypeStruct((_BATCH, 1), jnp.float32),
        compiler_params=pltpu.CompilerParams(
            dimension_semantics=("arbitrary",),
        ),
    )(uid, iid, u_emb, i_emb, utail, itail, W_lin, W1, W2, W3, b1, b2, scalars)


def kernel(user_ids, item_ids, user_table, item_table,
           W_lin, b_lin, W1, b1, W2, b2, W3, b3, alpha):
    utabT = user_table.T
    itabT = item_table.T
    u_emb, i_emb = _sc_gather(user_ids, item_ids, utabT, itabT)
    utail = lax.slice(user_table, (_ALIGNED, 0), (_NROWS, _EMB))
    itail = lax.slice(item_table, (_ALIGNED, 0), (_NROWS, _EMB))
    scalars = jnp.stack([b_lin[0], b3[0], alpha[0]]).reshape(1, 3)
    return _mlp_call(user_ids, item_ids, u_emb, i_emb, utail, itail,
                     W_lin, W1, W2, W3,
                     b1.reshape(1, _EMB), b2.reshape(1, 32), scalars)


# speedup vs baseline: 3.9570x; 1.9547x over previous
import functools

import jax
import jax.numpy as jnp
from jax import lax
from jax.experimental import pallas as pl
from jax.experimental.pallas import tpu as pltpu
from jax.experimental.pallas import tpu_sc as plsc

_BATCH = 16384
_EMB = 64
_NROWS = 1000000
_NSUP = 1952
_ALIGNED = _NSUP * 512
_NTAIL = _NROWS - _ALIGNED

_info = plsc.get_sparse_core_info()
_NC = _info.num_cores
_NS = _info.num_subcores
_NW = _NC * _NS
_CAP = 1024
_RING = 32

_sc_mesh = plsc.VectorSubcoreMesh(core_axis_name="c", subcore_axis_name="s")


def _splat(x, n=16):
    return jnp.full((n,), x, jnp.int32)


@functools.partial(
    pl.kernel,
    out_type=(
        jax.ShapeDtypeStruct((_BATCH, _EMB), jnp.float32),
        jax.ShapeDtypeStruct((_BATCH, _EMB), jnp.float32),
    ),
    mesh=_sc_mesh,
    scratch_types=[
        pltpu.VMEM((_BATCH,), jnp.int32),
        pltpu.VMEM((_CAP + 16,), jnp.int32),
        pltpu.VMEM((_CAP + 16,), jnp.int32),
        pltpu.VMEM((2, _EMB, 512), jnp.float32),
        pltpu.VMEM((32,), jnp.int32),
        pltpu.VMEM((32,), jnp.int32),
        pltpu.VMEM((_RING, _EMB), jnp.float32),
        pltpu.SemaphoreType.DMA,
        pltpu.SemaphoreType.DMA,
    ],
    compiler_params=pltpu.CompilerParams(needs_layout_passes=False),
)
def _sc_gather(uid_hbm, iid_hbm, utab_hbm, itab_hbm, u_out, i_out,
               idxv, hj, hr, pbuf, tj, tr, ring, gsem, osem):
    wid = lax.axis_index("s") * _NC + lax.axis_index("c")
    iota = lax.iota(jnp.int32, 16)

    def one_table(tab, ids_hbm, out, total_in):
        pltpu.sync_copy(ids_hbm, idxv)

        def scan_body(g, off):
            rv = idxv[pl.ds(g * 16, 16)]
            rv = jnp.where(rv >= _ALIGNED, 0, rv)
            mask = ((rv >> 9) & 31) == wid
            jv = iota + g * 16
            csum = plsc.cumsum(mask.astype(jnp.int32))
            pos = off + csum - 1
            plsc.store_scatter(hj, [pos], jv, mask=mask)
            plsc.store_scatter(hr, [pos], rv, mask=mask)
            return off + csum[15]

        nhits = lax.fori_loop(0, _BATCH // 16, scan_body, 0)
        ngroups = (nhits + 15) // 16
        npan = _NSUP // _NW

        def panel_start(k):
            return pl.multiple_of((wid + k * _NW) * 512, 128)

        def fire_panel(k, par):
            pltpu.async_copy(tab.at[:, pl.ds(panel_start(k), 512)],
                             pbuf.at[par], gsem)

        fire_panel(0, 0)

        def panel_body(k, total):
            par = lax.rem(k, 2)

            @pl.when(k + 1 < npan)
            def _():
                fire_panel(k + 1, 1 - par)

            pltpu.make_async_copy(tab.at[:, pl.ds(0, 512)],
                                  pbuf.at[0], gsem).wait()
            p = wid + k * _NW

            def group_body(m, total):
                rv = hr[pl.ds(m * 16, 16)]
                jv = hj[pl.ds(m * 16, 16)]
                lane = iota + m * 16
                msk = jnp.logical_and((rv >> 9) == p, lane < nhits)
                csum = plsc.cumsum(msk.astype(jnp.int32))
                cnt = csum[15]

                def extract(q, total):
                    j = tj[pl.ds(q, 16)][0]
                    rl = tr[pl.ds(q, 16)][0] & 511
                    slot = lax.rem(total, _RING)

                    @pl.when(total >= _RING)
                    def _():
                        pltpu.make_async_copy(
                            out.at[0], ring.at[0], osem).wait()
                    for c in range(_EMB // 16):
                        vals = plsc.load_gather(
                            pbuf, [_splat(par), iota + c * 16, _splat(rl)])
                        ring[slot, pl.ds(c * 16, 16)] = vals
                    pltpu.async_copy(ring.at[slot], out.at[j], osem)
                    return total + 1

                @pl.when(cnt > 0)
                def _():
                    pos = csum - 1
                    plsc.store_scatter(tj, [pos], jv, mask=msk)
                    plsc.store_scatter(tr, [pos], rv, mask=msk)

                return lax.cond(cnt > 0,
                                lambda t: lax.fori_loop(0, cnt, extract, t),
                                lambda t: t, total)

            return lax.fori_loop(0, ngroups, group_body, total)

        return lax.fori_loop(0, npan, panel_body, total_in)

    total = one_table(utab_hbm, uid_hbm, u_out, 0)
    total = one_table(itab_hbm, iid_hbm, i_out, total)

    def drain_body(q, _):
        pltpu.make_async_copy(u_out.at[0], ring.at[0], osem).wait()
        return _
    lax.fori_loop(0, jnp.minimum(total, _RING), drain_body, 0)


_ROWS = 2048


def _mlp_body(uid_ref, iid_ref, u_ref, i_ref, utail_ref, itail_ref,
              wlin_ref, w1_ref, w2_ref, w3_ref, b1_ref, b2_ref, scal_ref,
              o_ref):
    uid = uid_ref[...].reshape(_ROWS, 1)
    iid = iid_ref[...].reshape(_ROWS, 1)
    tail_iota = lax.broadcasted_iota(jnp.int32, (_ROWS, _NTAIL), 1)
    u_oh = (uid - _ALIGNED == tail_iota).astype(jnp.float32)
    i_oh = (iid - _ALIGNED == tail_iota).astype(jnp.float32)
    u_m = (uid >= _ALIGNED).astype(jnp.float32)
    i_m = (iid >= _ALIGNED).astype(jnp.float32)
    u = u_ref[...] * (1.0 - u_m) + jnp.dot(
        u_oh, utail_ref[...], preferred_element_type=jnp.float32)
    it = i_ref[...] * (1.0 - i_m) + jnp.dot(
        i_oh, itail_ref[...], preferred_element_type=jnp.float32)
    w1 = w1_ref[...]
    h1 = jnp.dot(u, w1[:_EMB], preferred_element_type=jnp.float32)
    h1 = h1 + jnp.dot(it, w1[_EMB:], preferred_element_type=jnp.float32)
    h1 = jnp.maximum(h1 + b1_ref[...], 0.0)
    h2 = jnp.dot(h1, w2_ref[...], preferred_element_type=jnp.float32)
    h2 = jnp.maximum(h2 + b2_ref[...], 0.0)
    neural = jnp.dot(h2, w3_ref[...], preferred_element_type=jnp.float32)
    wlin = wlin_ref[...]
    linear = jnp.dot(u, wlin[:_EMB], preferred_element_type=jnp.float32)
    linear = linear + jnp.dot(it, wlin[_EMB:], preferred_element_type=jnp.float32)
    b_lin = scal_ref[0, 0]
    b3 = scal_ref[0, 1]
    g = jax.nn.sigmoid(scal_ref[0, 2])
    o_ref[...] = g * (linear + b_lin) + (1.0 - g) * (neural + b3)


def _mlp_call(uid, iid, u_emb, i_emb, utail, itail,
              W_lin, W1, W2, W3, b1, b2, scalars):
    n_blocks = _BATCH // _ROWS
    full = lambda shape: pl.BlockSpec(shape, lambda i: (0,) * len(shape))
    return pl.pallas_call(
        _mlp_body,
        grid=(n_blocks,),
        in_specs=[
            pl.BlockSpec((_ROWS,), lambda i: (i,)),
            pl.BlockSpec((_ROWS,), lambda i: (i,)),
            pl.BlockSpec((_ROWS, _EMB), lambda i: (i, 0)),
            pl.BlockSpec((_ROWS, _EMB), lambda i: (i, 0)),
            full((_NTAIL, _EMB)),
            full((_NTAIL, _EMB)),
            full((2 * _EMB, 1)),
            full((2 * _EMB, _EMB)),
            full((_EMB, 32)),
            full((32, 1)),
            full((1, _EMB)),
            full((1, 32)),
            full((1, 3)),
        ],
        out_specs=pl.BlockSpec((_ROWS, 1), lambda i: (i, 0)),
        out_shape=jax.ShapeDtypeStruct((_BATCH, 1), jnp.float32),
        compiler_params=pltpu.CompilerParams(
            dimension_semantics=("arbitrary",),
        ),
    )(uid, iid, u_emb, i_emb, utail, itail, W_lin, W1, W2, W3, b1, b2, scalars)


def kernel(user_ids, item_ids, user_table, item_table,
           W_lin, b_lin, W1, b1, W2, b2, W3, b3, alpha):
    utabT = user_table.T
    itabT = item_table.T
    u_emb, i_emb = _sc_gather(user_ids, item_ids, utabT, itabT)
    utail = lax.slice(user_table, (_ALIGNED, 0), (_NROWS, _EMB))
    itail = lax.slice(item_table, (_ALIGNED, 0), (_NROWS, _EMB))
    scalars = jnp.stack([b_lin[0], b3[0], alpha[0]]).reshape(1, 3)
    return _mlp_call(user_ids, item_ids, u_emb, i_emb, utail, itail,
                     W_lin, W1, W2, W3,
                     b1.reshape(1, _EMB), b2.reshape(1, 32), scalars)


# speedup vs baseline: 4.3996x; 1.1119x over previous
import functools

import jax
import jax.numpy as jnp
from jax import lax
from jax.experimental import pallas as pl
from jax.experimental.pallas import tpu as pltpu
from jax.experimental.pallas import tpu_sc as plsc

_BATCH = 16384
_EMB = 64
_NROWS = 1000000
_NSUP = 1952
_ALIGNED = _NSUP * 512
_NTAIL = _NROWS - _ALIGNED

_info = plsc.get_sparse_core_info()
_NC = _info.num_cores
_NS = _info.num_subcores
_NW = _NC * _NS
_CAP = 1024
_RING = 32

_sc_mesh = plsc.VectorSubcoreMesh(core_axis_name="c", subcore_axis_name="s")


def _splat(x, n=16):
    return jnp.full((n,), x, jnp.int32)


@functools.partial(
    pl.kernel,
    out_type=(
        jax.ShapeDtypeStruct((_BATCH, _EMB), jnp.float32),
        jax.ShapeDtypeStruct((_BATCH, _EMB), jnp.float32),
    ),
    mesh=_sc_mesh,
    scratch_types=[
        pltpu.VMEM((_BATCH,), jnp.int32),
        pltpu.VMEM((_CAP + 16,), jnp.int32),
        pltpu.VMEM((_CAP + 16,), jnp.int32),
        pltpu.VMEM((4 * 384,), jnp.int32),
        pltpu.VMEM((4 * 384,), jnp.int32),
        pltpu.VMEM((2, _EMB, 512), jnp.float32),
        pltpu.VMEM((32,), jnp.int32),
        pltpu.VMEM((32,), jnp.int32),
        pltpu.VMEM((_RING, _EMB), jnp.float32),
        pltpu.SemaphoreType.DMA,
        pltpu.SemaphoreType.DMA,
    ],
    compiler_params=pltpu.CompilerParams(needs_layout_passes=False),
)
def _sc_gather(uid_hbm, iid_hbm, utab_hbm, itab_hbm, u_out, i_out,
               idxv, hj, hr, bj, br, pbuf, tj, tr, ring, gsem, osem):
    wid = lax.axis_index("s") * _NC + lax.axis_index("c")
    iota = lax.iota(jnp.int32, 16)

    def one_table(tab, ids_hbm, out, total_in):
        pltpu.sync_copy(ids_hbm, idxv)

        def scan_body(g, off):
            rv = idxv[pl.ds(g * 16, 16)]
            rv = jnp.where(rv >= _ALIGNED, 0, rv)
            mask = ((rv >> 9) & 31) == wid
            jv = iota + g * 16
            csum = plsc.cumsum(mask.astype(jnp.int32))
            pos = off + csum - 1
            plsc.store_scatter(hj, [pos], jv, mask=mask)
            plsc.store_scatter(hr, [pos], rv, mask=mask)
            return off + csum[15]

        nhits = lax.fori_loop(0, _BATCH // 16, scan_body, 0)
        npan = _NSUP // _NW

        def part_body(m, offs):
            rv = hr[pl.ds(m * 16, 16)]
            jv = hj[pl.ds(m * 16, 16)]
            lane = iota + m * 16
            kq = ((rv >> 9) - wid) >> 9
            valid = lane < nhits
            new_offs = []
            for q in range(4):
                msk = jnp.logical_and(kq == q, valid)
                cnt = plsc.all_reduce_population_count(msk)[0]
                key = jnp.where(msk, 0, 1)
                _, js = plsc.sort_key_val(key, jv)
                _, rs = plsc.sort_key_val(key, rv)
                bj[pl.ds(q * 384 + offs[q], 16)] = js
                br[pl.ds(q * 384 + offs[q], 16)] = rs
                new_offs.append(offs[q] + cnt)
            return tuple(new_offs)

        qcnt = lax.fori_loop(0, (nhits + 15) // 16, part_body, (0, 0, 0, 0))

        def panel_start(k):
            return pl.multiple_of((wid + k * _NW) * 512, 128)

        def fire_panel(k, par):
            pltpu.async_copy(tab.at[:, pl.ds(panel_start(k), 512)],
                             pbuf.at[par], gsem)

        fire_panel(0, 0)

        def panel_body(k, total):
            par = lax.rem(k, 2)

            @pl.when(k + 1 < npan)
            def _():
                fire_panel(k + 1, 1 - par)

            pltpu.make_async_copy(tab.at[:, pl.ds(0, 512)],
                                  pbuf.at[0], gsem).wait()
            p = wid + k * _NW
            q = k >> 4
            qbase = q * 384
            qn = jnp.where(q == 0, qcnt[0],
                           jnp.where(q == 1, qcnt[1],
                                     jnp.where(q == 2, qcnt[2], qcnt[3])))
            ngroups = (qn + 15) // 16

            def group_body(m, total):
                rv = br[pl.ds(qbase + m * 16, 16)]
                jv = bj[pl.ds(qbase + m * 16, 16)]
                lane = iota + m * 16
                msk = jnp.logical_and((rv >> 9) == p, lane < qn)
                csum = plsc.cumsum(msk.astype(jnp.int32))
                cnt = csum[15]

                def extract(q, total):
                    j = tj[pl.ds(q, 16)][0]
                    rl = tr[pl.ds(q, 16)][0] & 511
                    slot = lax.rem(total, _RING)

                    @pl.when(total >= _RING)
                    def _():
                        pltpu.make_async_copy(
                            out.at[0], ring.at[0], osem).wait()
                    for c in range(_EMB // 16):
                        vals = plsc.load_gather(
                            pbuf, [_splat(par), iota + c * 16, _splat(rl)])
                        ring[slot, pl.ds(c * 16, 16)] = vals
                    pltpu.async_copy(ring.at[slot], out.at[j], osem)
                    return total + 1

                @pl.when(cnt > 0)
                def _():
                    pos = csum - 1
                    plsc.store_scatter(tj, [pos], jv, mask=msk)
                    plsc.store_scatter(tr, [pos], rv, mask=msk)

                return lax.cond(cnt > 0,
                                lambda t: lax.fori_loop(0, cnt, extract, t),
                                lambda t: t, total)

            return lax.fori_loop(0, ngroups, group_body, total)

        return lax.fori_loop(0, npan, panel_body, total_in)

    total = one_table(utab_hbm, uid_hbm, u_out, 0)
    total = one_table(itab_hbm, iid_hbm, i_out, total)

    def drain_body(q, _):
        pltpu.make_async_copy(u_out.at[0], ring.at[0], osem).wait()
        return _
    lax.fori_loop(0, jnp.minimum(total, _RING), drain_body, 0)


_ROWS = 2048


def _mlp_body(uid_ref, iid_ref, u_ref, i_ref, utail_ref, itail_ref,
              wlin_ref, w1_ref, w2_ref, w3_ref, b1_ref, b2_ref, scal_ref,
              o_ref):
    uid = uid_ref[...].reshape(_ROWS, 1)
    iid = iid_ref[...].reshape(_ROWS, 1)
    tail_iota = lax.broadcasted_iota(jnp.int32, (_ROWS, _NTAIL), 1)
    u_oh = (uid - _ALIGNED == tail_iota).astype(jnp.float32)
    i_oh = (iid - _ALIGNED == tail_iota).astype(jnp.float32)
    u_m = (uid >= _ALIGNED).astype(jnp.float32)
    i_m = (iid >= _ALIGNED).astype(jnp.float32)
    u = u_ref[...] * (1.0 - u_m) + jnp.dot(
        u_oh, utail_ref[...], preferred_element_type=jnp.float32)
    it = i_ref[...] * (1.0 - i_m) + jnp.dot(
        i_oh, itail_ref[...], preferred_element_type=jnp.float32)
    w1 = w1_ref[...]
    h1 = jnp.dot(u, w1[:_EMB], preferred_element_type=jnp.float32)
    h1 = h1 + jnp.dot(it, w1[_EMB:], preferred_element_type=jnp.float32)
    h1 = jnp.maximum(h1 + b1_ref[...], 0.0)
    h2 = jnp.dot(h1, w2_ref[...], preferred_element_type=jnp.float32)
    h2 = jnp.maximum(h2 + b2_ref[...], 0.0)
    neural = jnp.dot(h2, w3_ref[...], preferred_element_type=jnp.float32)
    wlin = wlin_ref[...]
    linear = jnp.dot(u, wlin[:_EMB], preferred_element_type=jnp.float32)
    linear = linear + jnp.dot(it, wlin[_EMB:], preferred_element_type=jnp.float32)
    b_lin = scal_ref[0, 0]
    b3 = scal_ref[0, 1]
    g = jax.nn.sigmoid(scal_ref[0, 2])
    o_ref[...] = g * (linear + b_lin) + (1.0 - g) * (neural + b3)


def _mlp_call(uid, iid, u_emb, i_emb, utail, itail,
              W_lin, W1, W2, W3, b1, b2, scalars):
    n_blocks = _BATCH // _ROWS
    full = lambda shape: pl.BlockSpec(shape, lambda i: (0,) * len(shape))
    return pl.pallas_call(
        _mlp_body,
        grid=(n_blocks,),
        in_specs=[
            pl.BlockSpec((_ROWS,), lambda i: (i,)),
            pl.BlockSpec((_ROWS,), lambda i: (i,)),
            pl.BlockSpec((_ROWS, _EMB), lambda i: (i, 0)),
            pl.BlockSpec((_ROWS, _EMB), lambda i: (i, 0)),
            full((_NTAIL, _EMB)),
            full((_NTAIL, _EMB)),
            full((2 * _EMB, 1)),
            full((2 * _EMB, _EMB)),
            full((_EMB, 32)),
            full((32, 1)),
            full((1, _EMB)),
            full((1, 32)),
            full((1, 3)),
        ],
        out_specs=pl.BlockSpec((_ROWS, 1), lambda i: (i, 0)),
        out_shape=jax.ShapeDtypeStruct((_BATCH, 1), jnp.float32),
        compiler_params=pltpu.CompilerParams(
            dimension_semantics=("arbitrary",),
        ),
    )(uid, iid, u_emb, i_emb, utail, itail, W_lin, W1, W2, W3, b1, b2, scalars)


def kernel(user_ids, item_ids, user_table, item_table,
           W_lin, b_lin, W1, b1, W2, b2, W3, b3, alpha):
    utabT = user_table.T
    itabT = item_table.T
    u_emb, i_emb = _sc_gather(user_ids, item_ids, utabT, itabT)
    utail = lax.slice(user_table, (_ALIGNED, 0), (_NROWS, _EMB))
    itail = lax.slice(item_table, (_ALIGNED, 0), (_NROWS, _EMB))
    scalars = jnp.stack([b_lin[0], b3[0], alpha[0]]).reshape(1, 3)
    return _mlp_call(user_ids, item_ids, u_emb, i_emb, utail, itail,
                     W_lin, W1, W2, W3,
                     b1.reshape(1, _EMB), b2.reshape(1, 32), scalars)


# speedup vs baseline: 4.6712x; 1.0617x over previous
import functools

import jax
import jax.numpy as jnp
from jax import lax
from jax.experimental import pallas as pl
from jax.experimental.pallas import tpu as pltpu
from jax.experimental.pallas import tpu_sc as plsc

_BATCH = 16384
_EMB = 64
_NROWS = 1000000
_NSUP = 1952
_ALIGNED = _NSUP * 512
_NTAIL = _NROWS - _ALIGNED

_info = plsc.get_sparse_core_info()
_NC = _info.num_cores
_NS = _info.num_subcores
_NW = _NC * _NS
_CAP = 1024
_RING = 32

_sc_mesh = plsc.VectorSubcoreMesh(core_axis_name="c", subcore_axis_name="s")


def _splat(x, n=16):
    return jnp.full((n,), x, jnp.int32)


@functools.partial(
    pl.kernel,
    out_type=(
        jax.ShapeDtypeStruct((_BATCH, _EMB), jnp.float32),
        jax.ShapeDtypeStruct((_BATCH, _EMB), jnp.float32),
    ),
    mesh=_sc_mesh,
    scratch_types=[
        pltpu.VMEM((_BATCH,), jnp.int32),
        pltpu.VMEM((_CAP + 16,), jnp.int32),
        pltpu.VMEM((_CAP + 16,), jnp.int32),
        pltpu.VMEM((4 * 384,), jnp.int32),
        pltpu.VMEM((4 * 384,), jnp.int32),
        pltpu.VMEM((3, _EMB, 512), jnp.float32),
        pltpu.VMEM((32,), jnp.int32),
        pltpu.VMEM((32,), jnp.int32),
        pltpu.VMEM((_RING, _EMB), jnp.float32),
        pltpu.SemaphoreType.DMA,
        pltpu.SemaphoreType.DMA,
    ],
    compiler_params=pltpu.CompilerParams(needs_layout_passes=False),
)
def _sc_gather(uid_hbm, iid_hbm, utab_hbm, itab_hbm, u_out, i_out,
               idxv, hj, hr, bj, br, pbuf, tj, tr, ring, gsem, osem):
    wid = lax.axis_index("s") * _NC + lax.axis_index("c")
    iota = lax.iota(jnp.int32, 16)

    def one_table(tab, ids_hbm, out, total_in):
        pltpu.sync_copy(ids_hbm, idxv)

        def scan_body(g, off):
            rv = idxv[pl.ds(g * 16, 16)]
            rv = jnp.where(rv >= _ALIGNED, 0, rv)
            mask = ((rv >> 9) & 31) == wid
            jv = iota + g * 16
            csum = plsc.cumsum(mask.astype(jnp.int32))
            pos = off + csum - 1
            plsc.store_scatter(hj, [pos], jv, mask=mask)
            plsc.store_scatter(hr, [pos], rv, mask=mask)
            return off + csum[15]

        nhits = lax.fori_loop(0, _BATCH // 16, scan_body, 0)
        npan = _NSUP // _NW

        def part_body(m, offs):
            rv = hr[pl.ds(m * 16, 16)]
            jv = hj[pl.ds(m * 16, 16)]
            lane = iota + m * 16
            kq = ((rv >> 9) - wid) >> 9
            valid = lane < nhits
            new_offs = []
            for q in range(4):
                msk = jnp.logical_and(kq == q, valid)
                cnt = plsc.all_reduce_population_count(msk)[0]
                key = jnp.where(msk, 0, 1)
                _, js = plsc.sort_key_val(key, jv)
                _, rs = plsc.sort_key_val(key, rv)
                bj[pl.ds(q * 384 + offs[q], 16)] = js
                br[pl.ds(q * 384 + offs[q], 16)] = rs
                new_offs.append(offs[q] + cnt)
            return tuple(new_offs)

        qcnt = lax.fori_loop(0, (nhits + 15) // 16, part_body, (0, 0, 0, 0))

        def panel_start(k):
            return pl.multiple_of((wid + k * _NW) * 512, 128)

        def fire_panel(k, par):
            pltpu.async_copy(tab.at[:, pl.ds(panel_start(k), 512)],
                             pbuf.at[par], gsem)

        fire_panel(0, 0)
        fire_panel(1, 1)

        def panel_body(k, total):
            par = lax.rem(k, 3)

            @pl.when(k + 2 < npan)
            def _():
                fire_panel(k + 2, lax.rem(k + 2, 3))

            pltpu.make_async_copy(tab.at[:, pl.ds(0, 512)],
                                  pbuf.at[0], gsem).wait()
            p = wid + k * _NW
            q = k >> 4
            qbase = q * 384
            qn = jnp.where(q == 0, qcnt[0],
                           jnp.where(q == 1, qcnt[1],
                                     jnp.where(q == 2, qcnt[2], qcnt[3])))
            ngroups = (qn + 15) // 16

            def group_body(m, total):
                rv = br[pl.ds(qbase + m * 16, 16)]
                jv = bj[pl.ds(qbase + m * 16, 16)]
                lane = iota + m * 16
                msk = jnp.logical_and((rv >> 9) == p, lane < qn)
                csum = plsc.cumsum(msk.astype(jnp.int32))
                cnt = csum[15]

                def extract(q, total):
                    j = tj[pl.ds(q, 16)][0]
                    rl = tr[pl.ds(q, 16)][0] & 511
                    slot = lax.rem(total, _RING)

                    @pl.when(total >= _RING)
                    def _():
                        pltpu.make_async_copy(
                            out.at[0], ring.at[0], osem).wait()
                    for c in range(_EMB // 16):
                        vals = plsc.load_gather(
                            pbuf, [_splat(par), iota + c * 16, _splat(rl)])
                        ring[slot, pl.ds(c * 16, 16)] = vals
                    pltpu.async_copy(ring.at[slot], out.at[j], osem)
                    return total + 1

                @pl.when(cnt > 0)
                def _():
                    pos = csum - 1
                    plsc.store_scatter(tj, [pos], jv, mask=msk)
                    plsc.store_scatter(tr, [pos], rv, mask=msk)

                return lax.cond(cnt > 0,
                                lambda t: lax.fori_loop(0, cnt, extract, t),
                                lambda t: t, total)

            return lax.fori_loop(0, ngroups, group_body, total)

        return lax.fori_loop(0, npan, panel_body, total_in)

    total = one_table(utab_hbm, uid_hbm, u_out, 0)
    total = one_table(itab_hbm, iid_hbm, i_out, total)

    def drain_body(q, _):
        pltpu.make_async_copy(u_out.at[0], ring.at[0], osem).wait()
        return _
    lax.fori_loop(0, jnp.minimum(total, _RING), drain_body, 0)


_ROWS = 2048


def _mlp_body(uid_ref, iid_ref, u_ref, i_ref, utail_ref, itail_ref,
              wlin_ref, w1_ref, w2_ref, w3_ref, b1_ref, b2_ref, scal_ref,
              o_ref):
    uid = uid_ref[...].reshape(_ROWS, 1)
    iid = iid_ref[...].reshape(_ROWS, 1)
    tail_iota = lax.broadcasted_iota(jnp.int32, (_ROWS, _NTAIL), 1)
    u_oh = (uid - _ALIGNED == tail_iota).astype(jnp.float32)
    i_oh = (iid - _ALIGNED == tail_iota).astype(jnp.float32)
    u_m = (uid >= _ALIGNED).astype(jnp.float32)
    i_m = (iid >= _ALIGNED).astype(jnp.float32)
    u = u_ref[...] * (1.0 - u_m) + jnp.dot(
        u_oh, utail_ref[...], preferred_element_type=jnp.float32)
    it = i_ref[...] * (1.0 - i_m) + jnp.dot(
        i_oh, itail_ref[...], preferred_element_type=jnp.float32)
    w1 = w1_ref[...]
    h1 = jnp.dot(u, w1[:_EMB], preferred_element_type=jnp.float32)
    h1 = h1 + jnp.dot(it, w1[_EMB:], preferred_element_type=jnp.float32)
    h1 = jnp.maximum(h1 + b1_ref[...], 0.0)
    h2 = jnp.dot(h1, w2_ref[...], preferred_element_type=jnp.float32)
    h2 = jnp.maximum(h2 + b2_ref[...], 0.0)
    neural = jnp.dot(h2, w3_ref[...], preferred_element_type=jnp.float32)
    wlin = wlin_ref[...]
    linear = jnp.dot(u, wlin[:_EMB], preferred_element_type=jnp.float32)
    linear = linear + jnp.dot(it, wlin[_EMB:], preferred_element_type=jnp.float32)
    b_lin = scal_ref[0, 0]
    b3 = scal_ref[0, 1]
    g = jax.nn.sigmoid(scal_ref[0, 2])
    o_ref[...] = g * (linear + b_lin) + (1.0 - g) * (neural + b3)


def _mlp_call(uid, iid, u_emb, i_emb, utail, itail,
              W_lin, W1, W2, W3, b1, b2, scalars):
    n_blocks = _BATCH // _ROWS
    full = lambda shape: pl.BlockSpec(shape, lambda i: (0,) * len(shape))
    return pl.pallas_call(
        _mlp_body,
        grid=(n_blocks,),
        in_specs=[
            pl.BlockSpec((_ROWS,), lambda i: (i,)),
            pl.BlockSpec((_ROWS,), lambda i: (i,)),
            pl.BlockSpec((_ROWS, _EMB), lambda i: (i, 0)),
            pl.BlockSpec((_ROWS, _EMB), lambda i: (i, 0)),
            full((_NTAIL, _EMB)),
            full((_NTAIL, _EMB)),
            full((2 * _EMB, 1)),
            full((2 * _EMB, _EMB)),
            full((_EMB, 32)),
            full((32, 1)),
            full((1, _EMB)),
            full((1, 32)),
            full((1, 3)),
        ],
        out_specs=pl.BlockSpec((_ROWS, 1), lambda i: (i, 0)),
        out_shape=jax.ShapeDtypeStruct((_BATCH, 1), jnp.float32),
        compiler_params=pltpu.CompilerParams(
            dimension_semantics=("arbitrary",),
        ),
    )(uid, iid, u_emb, i_emb, utail, itail, W_lin, W1, W2, W3, b1, b2, scalars)


def kernel(user_ids, item_ids, user_table, item_table,
           W_lin, b_lin, W1, b1, W2, b2, W3, b3, alpha):
    utabT = user_table.T
    itabT = item_table.T
    u_emb, i_emb = _sc_gather(user_ids, item_ids, utabT, itabT)
    utail = lax.slice(user_table, (_ALIGNED, 0), (_NROWS, _EMB))
    itail = lax.slice(item_table, (_ALIGNED, 0), (_NROWS, _EMB))
    scalars = jnp.stack([b_lin[0], b3[0], alpha[0]]).reshape(1, 3)
    return _mlp_call(user_ids, item_ids, u_emb, i_emb, utail, itail,
                     W_lin, W1, W2, W3,
                     b1.reshape(1, _EMB), b2.reshape(1, 32), scalars)
